# Initial kernel scaffold; baseline (speedup 1.0000x reference)
import jax
import jax.numpy as jnp
from jax.experimental import pallas as pl


def kernel(cls_conv_out, rois):
    raise NotImplementedError("write your pallas kernel here")


# speedup vs baseline: 38.3135x; 38.3135x over previous
import functools

import jax
import jax.numpy as jnp
from jax import lax
from jax.experimental import pallas as pl
from jax.experimental.pallas import tpu as pltpu
from jax.experimental.pallas import tpu_sc as plsc

K = 7
CH = 1029
C1 = 21
H = W = 128
N = 300
NBLK = K * K
DPAD = 32
NIDX = 104
VL = 16


def _sat_body(x_ref, o_ref):
    x = x_ref[...]
    r = lax.broadcasted_iota(jnp.int32, (H, W), 0)
    c = lax.broadcasted_iota(jnp.int32, (H, W), 1)
    lower = (c < r).astype(jnp.float32)
    upper = (r < c).astype(jnp.float32)
    for ch in range(C1):
        t1 = jnp.dot(lower, x[ch], precision=lax.Precision.HIGHEST,
                     preferred_element_type=jnp.float32)
        o_ref[ch] = jnp.dot(t1, upper, precision=lax.Precision.HIGHEST,
                            preferred_element_type=jnp.float32)


def _make_sat(feat):
    return pl.pallas_call(
        _sat_body,
        grid=(NBLK,),
        in_specs=[pl.BlockSpec((C1, H, W), lambda i: (i, 0, 0))],
        out_specs=pl.BlockSpec((C1, H, W), lambda i: (i, 0, 0)),
        out_shape=jax.ShapeDtypeStruct((CH, H, W), jnp.float32),
    )(feat)


def _sc_pool(table, idx, scales, num_cores, num_subcores):
    nw = num_cores * num_subcores
    iters = -(-N // nw)
    mesh = plsc.VectorSubcoreMesh(core_axis_name="c", subcore_axis_name="s")

    @functools.partial(
        pl.kernel,
        mesh=mesh,
        compiler_params=pltpu.CompilerParams(use_tc_tiling_on_sc=False),
        out_type=jax.ShapeDtypeStruct((N, DPAD), jnp.float32),
        scratch_types=[
            pltpu.VMEM((NIDX,), jnp.int32),
            pltpu.VMEM((NIDX,), jnp.int32),
            pltpu.VMEM((NIDX, DPAD), jnp.float32),
            pltpu.VMEM((NIDX, DPAD), jnp.float32),
            pltpu.VMEM((VL,), jnp.float32),
            pltpu.VMEM((DPAD,), jnp.float32),
            pltpu.SemaphoreType.DMA,
            pltpu.SemaphoreType.DMA,
        ],
    )
    def kern(t_hbm, idx_hbm, sc_hbm, out_hbm,
             idx0_v, idx1_v, rows0_v, rows1_v, scale_v, out_v, sem0, sem1):
        wid = lax.axis_index("s") * num_cores + lax.axis_index("c")

        def body(i, carry):
            r = wid + i * nw

            @pl.when(r < N)
            def _():
                pltpu.sync_copy(idx_hbm.at[r, 0], idx0_v)
                pltpu.sync_copy(idx_hbm.at[r, 1], idx1_v)
                cp0 = pltpu.async_copy(t_hbm.at[idx0_v], rows0_v, sem0)
                cp1 = pltpu.async_copy(t_hbm.at[idx1_v], rows1_v, sem1)
                pltpu.sync_copy(sc_hbm.at[r], scale_v)
                cp0.wait()
                cp1.wait()
                acc0 = jnp.zeros((VL,), jnp.float32)
                acc1 = jnp.zeros((VL,), jnp.float32)
                for t in range(NBLK):
                    acc0 = (acc0 + rows0_v[t, 0:VL] - rows0_v[NBLK + t, 0:VL]
                            - rows1_v[t, 0:VL] + rows1_v[NBLK + t, 0:VL])
                    acc1 = (acc1 + rows0_v[t, VL:DPAD]
                            - rows0_v[NBLK + t, VL:DPAD]
                            - rows1_v[t, VL:DPAD]
                            + rows1_v[NBLK + t, VL:DPAD])
                s = scale_v[...]
                out_v[0:VL] = acc0 * s
                out_v[VL:DPAD] = acc1 * s
                pltpu.sync_copy(out_v, out_hbm.at[r])

            return carry

        lax.fori_loop(0, iters, body, 0)

    return kern(table, idx, scales)


def _softmax_body(x_ref, o_ref):
    x = x_ref[...]
    col = lax.broadcasted_iota(jnp.int32, (N, DPAD), 1)
    valid = col < C1
    xm = jnp.where(valid, x, -3e38)
    m = jnp.max(xm, axis=1, keepdims=True)
    e = jnp.where(valid, jnp.exp(x - m), 0.0)
    o_ref[...] = e / jnp.sum(e, axis=1, keepdims=True)


def _softmax(pooled):
    return pl.pallas_call(
        _softmax_body,
        out_shape=jax.ShapeDtypeStruct((N, DPAD), jnp.float32),
    )(pooled)


def kernel(cls_conv_out, rois):
    feat = cls_conv_out[cls_conv_out.shape[0] - 1]
    sat = _make_sat(feat)

    t4 = sat.reshape(NBLK, C1, H, W).transpose(0, 2, 3, 1)
    t4 = jnp.pad(t4, ((0, 0), (0, 0), (0, 0), (0, DPAD - C1)))
    table = t4.reshape(NBLK * H * W, DPAD)

    r32 = rois.astype(jnp.int32)
    ymin, xmin, ymax, xmax = r32[:, 0], r32[:, 1], r32[:, 2], r32[:, 3]
    ys = (ymax - ymin) // K
    xs = (xmax - xmin) // K
    jr = jnp.arange(K, dtype=jnp.int32)
    ay = ymin[:, None] + jr[None, :] * ys[:, None]
    by = ay + ys[:, None]
    cx = xmin[:, None] + jr[None, :] * xs[:, None]
    dx = cx + xs[:, None]
    jlbase = (jnp.arange(NBLK, dtype=jnp.int32) * (H * W)).reshape(K, K)

    def mk(yv, xv):
        v = jlbase[None, :, :] + yv[:, :, None] * W + xv[:, None, :]
        return v.reshape(N, NBLK)

    zpad = jnp.zeros((N, NIDX - 2 * NBLK), jnp.int32)
    g0 = jnp.concatenate([mk(by, dx), mk(ay, dx), zpad], axis=1)
    g1 = jnp.concatenate([mk(by, cx), mk(ay, cx), zpad], axis=1)
    idx = jnp.stack([g0, g1], axis=1)

    scales = (1.0 / (NBLK * ys.astype(jnp.float32) * xs.astype(jnp.float32)))
    scales = scales[:, None] * jnp.ones((1, VL), jnp.float32)

    info = plsc.get_sparse_core_info()
    pooled = _sc_pool(table, idx, scales, info.num_cores, info.num_subcores)
    out = _softmax(pooled)
    return out[:, :C1].reshape(N, C1, 1, 1)


# speedup vs baseline: 48.3631x; 1.2623x over previous
import functools

import jax
import jax.numpy as jnp
from jax import lax
from jax.experimental import pallas as pl
from jax.experimental.pallas import tpu as pltpu
from jax.experimental.pallas import tpu_sc as plsc

K = 7
CH = 1029
C1 = 21
H = W = 128
N = 300
NBLK = K * K
DPAD = 32
NIDX = 104
VL = 16


def _sat_body(x_ref, o_ref):
    x = x_ref[...]
    r = lax.broadcasted_iota(jnp.int32, (H, W), 0)
    c = lax.broadcasted_iota(jnp.int32, (H, W), 1)
    upper = (r < c).astype(jnp.float32)
    hp = lax.Precision.HIGHEST
    t1 = jnp.dot(x.reshape(C1 * H, W), upper, precision=hp,
                 preferred_element_type=jnp.float32)
    z = jnp.swapaxes(t1.reshape(C1, H, W), 1, 2)
    w = jnp.dot(z.reshape(C1 * W, H), upper, precision=hp,
                preferred_element_type=jnp.float32)
    wt = jnp.transpose(w.reshape(C1, W * H))
    zpad = jnp.zeros((W * H, DPAD - C1), jnp.float32)
    o_ref[...] = jnp.concatenate([wt, zpad], axis=1)


def _make_table(feat):
    return pl.pallas_call(
        _sat_body,
        grid=(NBLK,),
        in_specs=[pl.BlockSpec((C1, H, W), lambda i: (i, 0, 0))],
        out_specs=pl.BlockSpec((H * W, DPAD), lambda i: (i, 0)),
        out_shape=jax.ShapeDtypeStruct((NBLK * H * W, DPAD), jnp.float32),
        compiler_params=pltpu.CompilerParams(
            fuse_transposed_lhs_in_matmul=True),
    )(feat)


def _sc_pool(table, idx, scales, num_cores, num_subcores):
    nw = num_cores * num_subcores
    iters = -(-N // nw)
    mesh = plsc.VectorSubcoreMesh(core_axis_name="c", subcore_axis_name="s")

    @functools.partial(
        pl.kernel,
        mesh=mesh,
        compiler_params=pltpu.CompilerParams(use_tc_tiling_on_sc=False),
        out_type=jax.ShapeDtypeStruct((N, DPAD), jnp.float32),
        scratch_types=[
            pltpu.VMEM((NIDX,), jnp.int32),
            pltpu.VMEM((NIDX,), jnp.int32),
            pltpu.VMEM((NIDX, DPAD), jnp.float32),
            pltpu.VMEM((NIDX, DPAD), jnp.float32),
            pltpu.VMEM((VL,), jnp.float32),
            pltpu.VMEM((DPAD,), jnp.float32),
            pltpu.SemaphoreType.DMA,
            pltpu.SemaphoreType.DMA,
        ],
    )
    def kern(t_hbm, idx_hbm, sc_hbm, out_hbm,
             idx0_v, idx1_v, rows0_v, rows1_v, scale_v, out_v, sem0, sem1):
        wid = lax.axis_index("s") * num_cores + lax.axis_index("c")

        def body(i, carry):
            r = wid + i * nw

            @pl.when(r < N)
            def _():
                pltpu.sync_copy(idx_hbm.at[r, 0], idx0_v)
                pltpu.sync_copy(idx_hbm.at[r, 1], idx1_v)
                cp0 = pltpu.async_copy(t_hbm.at[idx0_v], rows0_v, sem0)
                cp1 = pltpu.async_copy(t_hbm.at[idx1_v], rows1_v, sem1)
                pltpu.sync_copy(sc_hbm.at[r], scale_v)
                cp0.wait()
                cp1.wait()
                acc0 = jnp.zeros((VL,), jnp.float32)
                acc1 = jnp.zeros((VL,), jnp.float32)
                for t in range(NBLK):
                    acc0 = (acc0 + rows0_v[t, 0:VL] - rows0_v[NBLK + t, 0:VL]
                            - rows1_v[t, 0:VL] + rows1_v[NBLK + t, 0:VL])
                    acc1 = (acc1 + rows0_v[t, VL:DPAD]
                            - rows0_v[NBLK + t, VL:DPAD]
                            - rows1_v[t, VL:DPAD]
                            + rows1_v[NBLK + t, VL:DPAD])
                s = scale_v[...]
                out_v[0:VL] = acc0 * s
                out_v[VL:DPAD] = acc1 * s
                pltpu.sync_copy(out_v, out_hbm.at[r])

            return carry

        lax.fori_loop(0, iters, body, 0)

    return kern(table, idx, scales)


def _softmax_body(x_ref, o_ref):
    x = x_ref[...]
    col = lax.broadcasted_iota(jnp.int32, (N, DPAD), 1)
    valid = col < C1
    xm = jnp.where(valid, x, -3e38)
    m = jnp.max(xm, axis=1, keepdims=True)
    e = jnp.where(valid, jnp.exp(x - m), 0.0)
    o_ref[...] = e / jnp.sum(e, axis=1, keepdims=True)


def _softmax(pooled):
    return pl.pallas_call(
        _softmax_body,
        out_shape=jax.ShapeDtypeStruct((N, DPAD), jnp.float32),
    )(pooled)


def kernel(cls_conv_out, rois):
    feat = cls_conv_out[cls_conv_out.shape[0] - 1]
    table = _make_table(feat)

    r32 = rois.astype(jnp.int32)
    ymin, xmin, ymax, xmax = r32[:, 0], r32[:, 1], r32[:, 2], r32[:, 3]
    ys = (ymax - ymin) // K
    xs = (xmax - xmin) // K
    jr = jnp.arange(K, dtype=jnp.int32)
    ay = ymin[:, None] + jr[None, :] * ys[:, None]
    by = ay + ys[:, None]
    cx = xmin[:, None] + jr[None, :] * xs[:, None]
    dx = cx + xs[:, None]
    jlbase = (jnp.arange(NBLK, dtype=jnp.int32) * (H * W)).reshape(K, K)

    def mk(yv, xv):
        v = jlbase[None, :, :] + xv[:, None, :] * H + yv[:, :, None]
        return v.reshape(N, NBLK)

    zpad = jnp.zeros((N, NIDX - 2 * NBLK), jnp.int32)
    g0 = jnp.concatenate([mk(by, dx), mk(ay, dx), zpad], axis=1)
    g1 = jnp.concatenate([mk(by, cx), mk(ay, cx), zpad], axis=1)
    idx = jnp.stack([g0, g1], axis=1)

    scales = (1.0 / (NBLK * ys.astype(jnp.float32) * xs.astype(jnp.float32)))
    scales = scales[:, None] * jnp.ones((1, VL), jnp.float32)

    info = plsc.get_sparse_core_info()
    pooled = _sc_pool(table, idx, scales, info.num_cores, info.num_subcores)
    out = _softmax(pooled)
    return out[:, :C1].reshape(N, C1, 1, 1)


# speedup vs baseline: 68.8424x; 1.4234x over previous
import functools

import jax
import jax.numpy as jnp
from jax import lax
from jax.experimental import pallas as pl
from jax.experimental.pallas import tpu as pltpu
from jax.experimental.pallas import tpu_sc as plsc

K = 7
CH = 1029
C1 = 21
H = W = 128
N = 300
NBLK = K * K
DPAD = 32
NIDX = 104
VL = 16


CB = 4
CPB = CB * C1
NQ = -(-NBLK // CB)


def _sat_body(x_ref, o_ref):
    x = x_ref[...]
    r = lax.broadcasted_iota(jnp.int32, (H, W), 0)
    c = lax.broadcasted_iota(jnp.int32, (H, W), 1)
    upper = (r < c).astype(jnp.float32)
    hp = lax.Precision.HIGHEST
    t1 = jnp.dot(x.reshape(CPB * H, W), upper, precision=hp,
                 preferred_element_type=jnp.float32)
    z = jnp.swapaxes(t1.reshape(CPB, H, W), 1, 2)
    w = jnp.dot(z.reshape(CPB * W, H), upper, precision=hp,
                preferred_element_type=jnp.float32)
    w2 = w.reshape(CPB, W * H)
    zrows = jnp.zeros((DPAD - C1, W * H), jnp.float32)
    pieces = []
    for b in range(CB):
        pieces.append(w2[b * C1:(b + 1) * C1])
        pieces.append(zrows)
    wp = jnp.concatenate(pieces, axis=0)
    o_ref[...] = jnp.transpose(wp)


def _make_table(feat):
    featp = jnp.pad(feat, ((0, NQ * CPB - CH), (0, 0), (0, 0)))
    return pl.pallas_call(
        _sat_body,
        grid=(NQ,),
        in_specs=[pl.BlockSpec((CPB, H, W), lambda i: (i, 0, 0))],
        out_specs=pl.BlockSpec((H * W, CB * DPAD), lambda i: (i, 0)),
        out_shape=jax.ShapeDtypeStruct((NQ * H * W, CB * DPAD),
                                       jnp.float32),
    )(featp)


def _sc_pool(table, idx, scales, num_cores, num_subcores):
    nw = num_cores * num_subcores
    iters = -(-N // nw)
    mesh = plsc.VectorSubcoreMesh(core_axis_name="c", subcore_axis_name="s")

    @functools.partial(
        pl.kernel,
        mesh=mesh,
        out_type=jax.ShapeDtypeStruct((N, DPAD), jnp.float32),
        scratch_types=[
            pltpu.VMEM((NIDX,), jnp.int32),
            pltpu.VMEM((NIDX,), jnp.int32),
            pltpu.VMEM((NIDX, CB * DPAD), jnp.float32),
            pltpu.VMEM((NIDX, CB * DPAD), jnp.float32),
            pltpu.VMEM((VL,), jnp.float32),
            pltpu.VMEM((DPAD,), jnp.float32),
            pltpu.SemaphoreType.DMA,
            pltpu.SemaphoreType.DMA,
        ],
    )
    def kern(t_hbm, idx_hbm, sc_hbm, out_hbm,
             idx0_v, idx1_v, rows0_v, rows1_v, scale_v, out_v, sem0, sem1):
        wid = lax.axis_index("s") * num_cores + lax.axis_index("c")

        def body(i, carry):
            r = wid + i * nw

            @pl.when(r < N)
            def _():
                pltpu.sync_copy(idx_hbm.at[r, 0], idx0_v)
                pltpu.sync_copy(idx_hbm.at[r, 1], idx1_v)
                cp0 = pltpu.async_copy(t_hbm.at[idx0_v], rows0_v, sem0)
                cp1 = pltpu.async_copy(t_hbm.at[idx1_v], rows1_v, sem1)
                pltpu.sync_copy(sc_hbm.at[r], scale_v)
                cp0.wait()
                cp1.wait()
                acc0 = jnp.zeros((VL,), jnp.float32)
                acc1 = jnp.zeros((VL,), jnp.float32)
                for t in range(NBLK):
                    lo = (t % CB) * DPAD
                    hi = lo + VL
                    acc0 = (acc0 + rows0_v[t, lo:hi]
                            - rows0_v[NBLK + t, lo:hi]
                            - rows1_v[t, lo:hi]
                            + rows1_v[NBLK + t, lo:hi])
                    acc1 = (acc1 + rows0_v[t, hi:hi + VL]
                            - rows0_v[NBLK + t, hi:hi + VL]
                            - rows1_v[t, hi:hi + VL]
                            + rows1_v[NBLK + t, hi:hi + VL])
                s = scale_v[...]
                out_v[0:VL] = acc0 * s
                out_v[VL:DPAD] = acc1 * s
                pltpu.sync_copy(out_v, out_hbm.at[r])

            return carry

        lax.fori_loop(0, iters, body, 0)

    return kern(table, idx, scales)


def _softmax_body(x_ref, o_ref):
    x = x_ref[...]
    col = lax.broadcasted_iota(jnp.int32, (N, DPAD), 1)
    valid = col < C1
    xm = jnp.where(valid, x, -3e38)
    m = jnp.max(xm, axis=1, keepdims=True)
    e = jnp.where(valid, jnp.exp(x - m), 0.0)
    o_ref[...] = e / jnp.sum(e, axis=1, keepdims=True)


def _softmax(pooled):
    return pl.pallas_call(
        _softmax_body,
        out_shape=jax.ShapeDtypeStruct((N, DPAD), jnp.float32),
    )(pooled)


def kernel(cls_conv_out, rois):
    feat = cls_conv_out[cls_conv_out.shape[0] - 1]
    table = _make_table(feat)

    r32 = rois.astype(jnp.int32)
    ymin, xmin, ymax, xmax = r32[:, 0], r32[:, 1], r32[:, 2], r32[:, 3]
    ys = (ymax - ymin) // K
    xs = (xmax - xmin) // K
    jr = jnp.arange(K, dtype=jnp.int32)
    ay = ymin[:, None] + jr[None, :] * ys[:, None]
    by = ay + ys[:, None]
    cx = xmin[:, None] + jr[None, :] * xs[:, None]
    dx = cx + xs[:, None]
    jlbase = ((jnp.arange(NBLK, dtype=jnp.int32) // CB)
              * (H * W)).reshape(K, K)

    def mk(yv, xv):
        v = jlbase[None, :, :] + xv[:, None, :] * H + yv[:, :, None]
        return v.reshape(N, NBLK)

    zpad = jnp.zeros((N, NIDX - 2 * NBLK), jnp.int32)
    g0 = jnp.concatenate([mk(by, dx), mk(ay, dx), zpad], axis=1)
    g1 = jnp.concatenate([mk(by, cx), mk(ay, cx), zpad], axis=1)
    idx = jnp.stack([g0, g1], axis=1)

    scales = (1.0 / (NBLK * ys.astype(jnp.float32) * xs.astype(jnp.float32)))
    scales = scales[:, None] * jnp.ones((1, VL), jnp.float32)

    info = plsc.get_sparse_core_info()
    pooled = _sc_pool(table, idx, scales, info.num_cores, info.num_subcores)
    out = _softmax(pooled)
    return out[:, :C1].reshape(N, C1, 1, 1)
